# baseline (speedup 1.0000x reference)
import functools

import jax
import jax.numpy as jnp
from jax import lax
from jax.experimental import pallas as pl
from jax.experimental.pallas import tpu as pltpu
from jax.experimental.pallas import tpu_sc as plsc

_NC, _NS = 2, 16
_NW = _NC * _NS
_L = 16
_CHUNK = 128


def _lane_sum(v):
    lanes = lax.iota(jnp.int32, _L)
    for sh in (8, 4, 2, 1):
        perm = lanes ^ sh
        v = v + v.at[perm].get(mode="promise_in_bounds", unique_indices=True)
    return v


def _rsqrt(x):
    i = lax.bitcast_convert_type(x, jnp.int32)
    i = jnp.int32(0x5F3759DF) - (i >> 1)
    y = lax.bitcast_convert_type(i, jnp.float32)
    h = x * jnp.float32(0.5)
    y = y * (jnp.float32(1.5) - h * y * y)
    y = y * (jnp.float32(1.5) - h * y * y)
    y = y * (jnp.float32(1.5) - h * y * y)
    return y


def kernel(x, word_table, pos_table, gamma, beta):
    B, S = x.shape
    V, D = word_table.shape
    N = B * S
    n_per_w = N // _NW
    n_chunks = n_per_w // _CHUNK
    assert n_per_w * _NW == N and n_chunks * _CHUNK == n_per_w
    assert n_per_w % S == 0
    xf = x.reshape(N)

    mesh = plsc.VectorSubcoreMesh(core_axis_name="c", subcore_axis_name="s")

    @functools.partial(
        pl.kernel,
        out_type=jax.ShapeDtypeStruct((N, D), jnp.float32),
        mesh=mesh,
        scratch_types=[
            pltpu.VMEM((_CHUNK,), jnp.int32),
            pltpu.VMEM((_CHUNK, D), jnp.float32),
            pltpu.VMEM((S, D), jnp.float32),
            pltpu.VMEM((D,), jnp.float32),
            pltpu.VMEM((D,), jnp.float32),
            pltpu.SemaphoreType.DMA,
        ],
        compiler_params=pltpu.CompilerParams(use_tc_tiling_on_sc=False),
    )
    def sc_kernel(x_hbm, word_hbm, pos_hbm, gamma_hbm, beta_hbm, out_hbm,
                  idx_v, rows_v, pos_v, gamma_v, beta_v, sem):
        wid = lax.axis_index("s") * _NC + lax.axis_index("c")
        base = wid * n_per_w
        pltpu.sync_copy(pos_hbm, pos_v)
        pltpu.sync_copy(gamma_hbm, gamma_v)
        pltpu.sync_copy(beta_hbm, beta_v)

        def chunk_body(c, carry):
            off = c * _CHUNK
            gbase = base + off
            pltpu.sync_copy(x_hbm.at[pl.ds(gbase, _CHUNK)], idx_v)
            pltpu.async_copy(word_hbm.at[idx_v], rows_v, sem).wait()

            def row_body(i, carry2):
                p = (off + i) % S
                vs = []
                acc_s = None
                acc_q = None
                for j in range(D // _L):
                    v = rows_v[i, pl.ds(j * _L, _L)] + pos_v[p, pl.ds(j * _L, _L)]
                    vs.append(v)
                    acc_s = v if acc_s is None else acc_s + v
                    acc_q = v * v if acc_q is None else acc_q + v * v
                mean = _lane_sum(acc_s) * jnp.float32(1.0 / D)
                var = _lane_sum(acc_q) * jnp.float32(1.0 / D) - mean * mean
                scale = _rsqrt(var + jnp.float32(1e-5))
                for j in range(D // _L):
                    g = gamma_v[pl.ds(j * _L, _L)]
                    bb = beta_v[pl.ds(j * _L, _L)]
                    rows_v[i, pl.ds(j * _L, _L)] = (vs[j] - mean) * scale * g + bb
                return carry2

            lax.fori_loop(0, _CHUNK, row_body, 0)
            pltpu.sync_copy(rows_v, out_hbm.at[pl.ds(gbase, _CHUNK)])
            return carry

        lax.fori_loop(0, n_chunks, chunk_body, 0)

    out = sc_kernel(xf, word_table, pos_table, gamma, beta)
    return out.reshape(B, S, D)


# speedup vs baseline: 1.7725x; 1.7725x over previous
import functools

import jax
import jax.numpy as jnp
from jax import lax
from jax.experimental import pallas as pl
from jax.experimental.pallas import tpu as pltpu
from jax.experimental.pallas import tpu_sc as plsc

_NC, _NS = 2, 16
_NW = _NC * _NS
_GCHUNK = 128
_RING_ROWS = 512
_NBUF = 2
_BLK = 3200


def _sc_gather(xf, word_table):
    N = xf.shape[0]
    V, D = word_table.shape
    n_per_w = N // _NW
    n_chunks = n_per_w // _RING_ROWS
    assert n_per_w * _NW == N and n_chunks * _RING_ROWS == n_per_w
    assert n_chunks % _NBUF == 0
    n_gath = _RING_ROWS // _GCHUNK
    chunk_bytes = _RING_ROWS * D * 4

    mesh = plsc.VectorSubcoreMesh(core_axis_name="c", subcore_axis_name="s")

    @functools.partial(
        pl.kernel,
        out_type=jax.ShapeDtypeStruct((N, D), jnp.float32),
        mesh=mesh,
        scratch_types=[
            pltpu.VMEM((_NBUF, _RING_ROWS), jnp.int32),
            pltpu.VMEM((_NBUF, _RING_ROWS, D), jnp.float32),
            pltpu.SemaphoreType.DMA,
            pltpu.SemaphoreType.DMA,
        ],
        compiler_params=pltpu.CompilerParams(use_tc_tiling_on_sc=False),
    )
    def gather_kernel(x_hbm, word_hbm, out_hbm, idx_v, rows_v, sem0, sem1):
        wid = lax.axis_index("s") * _NC + lax.axis_index("c")
        base = wid * n_per_w
        sems = (sem0, sem1)

        def fire(buf, step):
            off = base + step * _RING_ROWS
            pltpu.sync_copy(x_hbm.at[pl.ds(off, _RING_ROWS)], idx_v.at[buf])
            for j in range(n_gath):
                pltpu.async_copy(
                    word_hbm.at[idx_v.at[buf, pl.ds(j * _GCHUNK, _GCHUNK)]],
                    rows_v.at[buf, pl.ds(j * _GCHUNK, _GCHUNK)],
                    sems[buf],
                )

        def drain_write(buf, step):
            off = base + step * _RING_ROWS
            pltpu.make_async_copy(
                word_hbm.at[pl.ds(0, _RING_ROWS)], rows_v.at[buf], sems[buf]
            ).wait()
            pltpu.sync_copy(rows_v.at[buf], out_hbm.at[pl.ds(off, _RING_ROWS)])

        fire(0, 0)

        def outer(i, carry):
            s0 = i * _NBUF
            fire(1, s0 + 1)
            drain_write(0, s0)

            @pl.when(s0 + 2 < n_chunks)
            def _():
                fire(0, s0 + 2)

            drain_write(1, s0 + 1)
            return carry

        lax.fori_loop(0, n_chunks // _NBUF, outer, 0)

    return gather_kernel(xf, word_table)


def _ln_body(g_ref, pos_ref, gamma_ref, beta_ref, o_ref):
    h = g_ref[...] + pos_ref[...]
    mean = jnp.mean(h, axis=-1, keepdims=True)
    d = h - mean
    var = jnp.mean(d * d, axis=-1, keepdims=True)
    o_ref[...] = d * lax.rsqrt(var + jnp.float32(1e-5)) * gamma_ref[...] + beta_ref[...]


def _tc_layernorm(g, pos_tiled, gamma, beta):
    N, D = g.shape
    n_blocks = N // _BLK
    assert n_blocks * _BLK == N
    return pl.pallas_call(
        _ln_body,
        grid=(n_blocks,),
        in_specs=[
            pl.BlockSpec((_BLK, D), lambda i: (i, 0)),
            pl.BlockSpec((_BLK, D), lambda i: (0, 0)),
            pl.BlockSpec((1, D), lambda i: (0, 0)),
            pl.BlockSpec((1, D), lambda i: (0, 0)),
        ],
        out_specs=pl.BlockSpec((_BLK, D), lambda i: (i, 0)),
        out_shape=jax.ShapeDtypeStruct((N, D), jnp.float32),
    )(g, pos_tiled, gamma, beta)


def kernel(x, word_table, pos_table, gamma, beta):
    B, S = x.shape
    V, D = word_table.shape
    N = B * S
    assert _BLK % S == 0
    xf = x.reshape(N)
    g = _sc_gather(xf, word_table)
    pos_tiled = jnp.tile(pos_table, (_BLK // S, 1))
    out = _tc_layernorm(g, pos_tiled, gamma.reshape(1, D), beta.reshape(1, D))
    return out.reshape(B, S, D)


# speedup vs baseline: 2.0100x; 1.1340x over previous
import functools

import jax
import jax.numpy as jnp
from jax import lax
from jax.experimental import pallas as pl
from jax.experimental.pallas import tpu as pltpu
from jax.experimental.pallas import tpu_sc as plsc

_NC, _NS = 2, 16
_NW = _NC * _NS
_GCHUNK = 128
_RING_ROWS = 512
_NBUF = 2


def _sc_gather(xf, word_table):
    N = xf.shape[0]
    V, D = word_table.shape
    n_per_w = N // _NW
    n_chunks = n_per_w // _RING_ROWS
    assert n_per_w * _NW == N and n_chunks * _RING_ROWS == n_per_w
    assert n_chunks % _NBUF == 0
    n_gath = _RING_ROWS // _GCHUNK

    mesh = plsc.VectorSubcoreMesh(core_axis_name="c", subcore_axis_name="s")

    @functools.partial(
        pl.kernel,
        out_type=jax.ShapeDtypeStruct((N, D), jnp.float32),
        mesh=mesh,
        scratch_types=[
            pltpu.VMEM((_NBUF, _RING_ROWS), jnp.int32),
            pltpu.VMEM((_NBUF, _RING_ROWS, D), jnp.float32),
            pltpu.SemaphoreType.DMA,
            pltpu.SemaphoreType.DMA,
        ],
        compiler_params=pltpu.CompilerParams(use_tc_tiling_on_sc=False),
    )
    def gather_kernel(x_hbm, word_hbm, out_hbm, idx_v, rows_v, sem0, sem1):
        wid = lax.axis_index("s") * _NC + lax.axis_index("c")
        base = wid * n_per_w
        sems = (sem0, sem1)

        def fire(buf, step):
            off = base + step * _RING_ROWS
            pltpu.sync_copy(x_hbm.at[pl.ds(off, _RING_ROWS)], idx_v.at[buf])
            for j in range(n_gath):
                pltpu.async_copy(
                    word_hbm.at[idx_v.at[buf, pl.ds(j * _GCHUNK, _GCHUNK)]],
                    rows_v.at[buf, pl.ds(j * _GCHUNK, _GCHUNK)],
                    sems[buf],
                )

        def drain_write(buf, step):
            off = base + step * _RING_ROWS
            pltpu.make_async_copy(
                word_hbm.at[pl.ds(0, _RING_ROWS)], rows_v.at[buf], sems[buf]
            ).wait()
            pltpu.sync_copy(rows_v.at[buf], out_hbm.at[pl.ds(off, _RING_ROWS)])

        fire(0, 0)

        def outer(i, carry):
            s0 = i * _NBUF
            fire(1, s0 + 1)
            drain_write(0, s0)

            @pl.when(s0 + 2 < n_chunks)
            def _():
                fire(0, s0 + 2)

            drain_write(1, s0 + 1)
            return carry

        lax.fori_loop(0, n_chunks // _NBUF, outer, 0)

    return gather_kernel(xf, word_table)


def _ln_body(g_ref, pos_ref, gamma_ref, beta_ref, o_ref):
    h = g_ref[0] + pos_ref[0]
    mean = jnp.mean(h, axis=-1, keepdims=True)
    d = h - mean
    var = jnp.mean(d * d, axis=-1, keepdims=True)
    o = d * lax.rsqrt(var + jnp.float32(1e-5)) * gamma_ref[...] + beta_ref[...]
    o_ref[0] = o.T


def _tc_layernorm(g3, posP, gamma, beta):
    S, B, D = g3.shape
    return pl.pallas_call(
        _ln_body,
        grid=(S,),
        in_specs=[
            pl.BlockSpec((1, B, D), lambda s: (s, 0, 0)),
            pl.BlockSpec((1, 1, D), lambda s: (s, 0, 0)),
            pl.BlockSpec((1, D), lambda s: (0, 0)),
            pl.BlockSpec((1, D), lambda s: (0, 0)),
        ],
        out_specs=pl.BlockSpec((1, D, B), lambda s: (s, 0, 0)),
        out_shape=jax.ShapeDtypeStruct((S, D, B), jnp.float32),
    )(g3, posP, gamma, beta)


def kernel(x, word_table, pos_table, gamma, beta):
    B, S = x.shape
    V, D = word_table.shape
    N = B * S
    xT = jnp.swapaxes(x, 0, 1).reshape(N)
    g = _sc_gather(xT, word_table)
    out_phys = _tc_layernorm(
        g.reshape(S, B, D),
        pos_table.reshape(S, 1, D),
        gamma.reshape(1, D),
        beta.reshape(1, D),
    )
    return jnp.transpose(out_phys, (2, 0, 1))


# speedup vs baseline: 2.1581x; 1.0737x over previous
import functools

import jax
import jax.numpy as jnp
from jax import lax
from jax.experimental import pallas as pl
from jax.experimental.pallas import tpu as pltpu
from jax.experimental.pallas import tpu_sc as plsc

_NC, _NS = 2, 16
_NW = _NC * _NS
_GCHUNK = 128
_RING_ROWS = 640
_NBUF = 2


def _sc_gather(xf, word_table):
    N = xf.shape[0]
    V, D = word_table.shape
    n_per_w = N // _NW
    n_chunks = n_per_w // _RING_ROWS
    assert n_per_w * _NW == N and n_chunks * _RING_ROWS == n_per_w
    assert n_chunks % _NBUF == 0
    n_gath = _RING_ROWS // _GCHUNK

    mesh = plsc.VectorSubcoreMesh(core_axis_name="c", subcore_axis_name="s")

    @functools.partial(
        pl.kernel,
        out_type=jax.ShapeDtypeStruct((N, D), jnp.float32),
        mesh=mesh,
        scratch_types=[
            pltpu.VMEM((n_per_w,), jnp.int32),
            pltpu.VMEM((_NBUF, _RING_ROWS, D), jnp.float32),
            pltpu.SemaphoreType.DMA,
            pltpu.SemaphoreType.DMA,
            pltpu.SemaphoreType.DMA,
            pltpu.SemaphoreType.DMA,
        ],
        compiler_params=pltpu.CompilerParams(use_tc_tiling_on_sc=False),
    )
    def gather_kernel(x_hbm, word_hbm, out_hbm, idx_v, rows_v,
                      semg0, semg1, semw0, semw1):
        wid = lax.axis_index("s") * _NC + lax.axis_index("c")
        base = wid * n_per_w
        semg = (semg0, semg1)
        semw = (semw0, semw1)

        pltpu.sync_copy(x_hbm.at[pl.ds(base, n_per_w)], idx_v)

        def fire(buf, step):
            off = step * _RING_ROWS
            for j in range(n_gath):
                pltpu.async_copy(
                    word_hbm.at[idx_v.at[pl.ds(off + j * _GCHUNK, _GCHUNK)]],
                    rows_v.at[buf, pl.ds(j * _GCHUNK, _GCHUNK)],
                    semg[buf],
                )

        def drain_g(buf):
            pltpu.make_async_copy(
                word_hbm.at[pl.ds(0, _RING_ROWS)], rows_v.at[buf], semg[buf]
            ).wait()

        def write_async(buf, step):
            pltpu.async_copy(
                rows_v.at[buf],
                out_hbm.at[pl.ds(base + step * _RING_ROWS, _RING_ROWS)],
                semw[buf],
            )

        def wait_w(buf):
            pltpu.make_async_copy(
                rows_v.at[buf], out_hbm.at[pl.ds(0, _RING_ROWS)], semw[buf]
            ).wait()

        fire(0, 0)

        def outer(i, carry):
            s0 = i * _NBUF

            @pl.when(i > 0)
            def _():
                wait_w(1)

            fire(1, s0 + 1)
            drain_g(0)
            write_async(0, s0)

            @pl.when(s0 + 2 < n_chunks)
            def _():
                wait_w(0)
                fire(0, s0 + 2)

            drain_g(1)
            write_async(1, s0 + 1)
            return carry

        lax.fori_loop(0, n_chunks // _NBUF, outer, 0)
        wait_w(0)
        wait_w(1)

    return gather_kernel(xf, word_table)


def _ln_body(g_ref, pos_ref, gamma_ref, beta_ref, o_ref):
    h = g_ref[0].T + pos_ref[0]
    mean = jnp.mean(h, axis=0, keepdims=True)
    d = h - mean
    var = jnp.mean(d * d, axis=0, keepdims=True)
    o_ref[0] = d * lax.rsqrt(var + jnp.float32(1e-5)) * gamma_ref[...] + beta_ref[...]


def _tc_layernorm(g3, posP, gammaT, betaT):
    S, B, D = g3.shape
    return pl.pallas_call(
        _ln_body,
        grid=(S,),
        in_specs=[
            pl.BlockSpec((1, B, D), lambda s: (s, 0, 0)),
            pl.BlockSpec((1, D, 1), lambda s: (s, 0, 0)),
            pl.BlockSpec((D, 1), lambda s: (0, 0)),
            pl.BlockSpec((D, 1), lambda s: (0, 0)),
        ],
        out_specs=pl.BlockSpec((1, D, B), lambda s: (s, 0, 0)),
        out_shape=jax.ShapeDtypeStruct((S, D, B), jnp.float32),
    )(g3, posP, gammaT, betaT)


def kernel(x, word_table, pos_table, gamma, beta):
    B, S = x.shape
    V, D = word_table.shape
    N = B * S
    xT = jnp.swapaxes(x, 0, 1).reshape(N)
    g = _sc_gather(xT, word_table)
    out_phys = _tc_layernorm(
        g.reshape(S, B, D),
        pos_table.reshape(S, D, 1),
        gamma.reshape(D, 1),
        beta.reshape(D, 1),
    )
    return jnp.transpose(out_phys, (2, 0, 1))


# speedup vs baseline: 2.1974x; 1.0182x over previous
import functools

import jax
import jax.numpy as jnp
from jax import lax
from jax.experimental import pallas as pl
from jax.experimental.pallas import tpu as pltpu
from jax.experimental.pallas import tpu_sc as plsc

_NC, _NS = 2, 16
_NW = _NC * _NS
_GCHUNK = 128
_RING_ROWS = 640
_NBUF = 2


def _sc_gather(xf, word_table):
    N = xf.shape[0]
    V, D = word_table.shape
    n_per_w = N // _NW
    n_chunks = n_per_w // _RING_ROWS
    assert n_per_w * _NW == N and n_chunks * _RING_ROWS == n_per_w
    assert n_chunks % _NBUF == 0
    n_gath = _RING_ROWS // _GCHUNK

    mesh = plsc.VectorSubcoreMesh(core_axis_name="c", subcore_axis_name="s")

    @functools.partial(
        pl.kernel,
        out_type=jax.ShapeDtypeStruct((N, D), jnp.float32),
        mesh=mesh,
        scratch_types=[
            pltpu.VMEM((n_per_w,), jnp.int32),
            pltpu.VMEM((_NBUF, _RING_ROWS, D), jnp.float32),
            pltpu.SemaphoreType.DMA,
            pltpu.SemaphoreType.DMA,
            pltpu.SemaphoreType.DMA,
            pltpu.SemaphoreType.DMA,
        ],
        compiler_params=pltpu.CompilerParams(use_tc_tiling_on_sc=False),
    )
    def gather_kernel(x_hbm, word_hbm, out_hbm, idx_v, rows_v,
                      semg0, semg1, semw0, semw1):
        wid = lax.axis_index("s") * _NC + lax.axis_index("c")
        base = wid * n_per_w
        semg = (semg0, semg1)
        semw = (semw0, semw1)

        pltpu.sync_copy(x_hbm.at[pl.ds(base, n_per_w)], idx_v)

        def fire(buf, step):
            off = step * _RING_ROWS
            for j in range(n_gath):
                pltpu.async_copy(
                    word_hbm.at[idx_v.at[pl.ds(off + j * _GCHUNK, _GCHUNK)]],
                    rows_v.at[buf, pl.ds(j * _GCHUNK, _GCHUNK)],
                    semg[buf],
                )

        def drain_g(buf):
            pltpu.make_async_copy(
                word_hbm.at[pl.ds(0, _RING_ROWS)], rows_v.at[buf], semg[buf]
            ).wait()

        def write_async(buf, step):
            pltpu.async_copy(
                rows_v.at[buf],
                out_hbm.at[pl.ds(base + step * _RING_ROWS, _RING_ROWS)],
                semw[buf],
            )

        def wait_w(buf):
            pltpu.make_async_copy(
                rows_v.at[buf], out_hbm.at[pl.ds(0, _RING_ROWS)], semw[buf]
            ).wait()

        fire(0, 0)

        def outer(i, carry):
            s0 = i * _NBUF

            @pl.when(i > 0)
            def _():
                wait_w(1)

            fire(1, s0 + 1)
            drain_g(0)
            write_async(0, s0)

            @pl.when(s0 + 2 < n_chunks)
            def _():
                wait_w(0)
                fire(0, s0 + 2)

            drain_g(1)
            write_async(1, s0 + 1)
            return carry

        lax.fori_loop(0, n_chunks // _NBUF, outer, 0)
        wait_w(0)
        wait_w(1)

    return gather_kernel(xf, word_table)


_SBLK = 2


def _ln_body(g_ref, pos_ref, gamma_ref, beta_ref, o_ref):
    for k in range(_SBLK):
        h = g_ref[k].T + pos_ref[k]
        mean = jnp.mean(h, axis=0, keepdims=True)
        d = h - mean
        var = jnp.mean(d * d, axis=0, keepdims=True)
        o_ref[k] = (
            d * lax.rsqrt(var + jnp.float32(1e-5)) * gamma_ref[...] + beta_ref[...]
        )


def _tc_layernorm(g3, posP, gammaT, betaT):
    S, B, D = g3.shape
    return pl.pallas_call(
        _ln_body,
        grid=(S // _SBLK,),
        in_specs=[
            pl.BlockSpec((_SBLK, B, D), lambda s: (s, 0, 0)),
            pl.BlockSpec((_SBLK, D, 1), lambda s: (s, 0, 0)),
            pl.BlockSpec((D, 1), lambda s: (0, 0)),
            pl.BlockSpec((D, 1), lambda s: (0, 0)),
        ],
        out_specs=pl.BlockSpec((_SBLK, D, B), lambda s: (s, 0, 0)),
        out_shape=jax.ShapeDtypeStruct((S, D, B), jnp.float32),
    )(g3, posP, gammaT, betaT)


def kernel(x, word_table, pos_table, gamma, beta):
    B, S = x.shape
    V, D = word_table.shape
    N = B * S
    xT = jnp.swapaxes(x, 0, 1).reshape(N)
    g = _sc_gather(xT, word_table)
    out_phys = _tc_layernorm(
        g.reshape(S, B, D),
        pos_table.reshape(S, D, 1),
        gamma.reshape(D, 1),
        beta.reshape(D, 1),
    )
    return jnp.transpose(out_phys, (2, 0, 1))


# speedup vs baseline: 2.2310x; 1.0153x over previous
import functools

import jax
import jax.numpy as jnp
from jax import lax
from jax.experimental import pallas as pl
from jax.experimental.pallas import tpu as pltpu
from jax.experimental.pallas import tpu_sc as plsc

_NC, _NS = 2, 16
_NW = _NC * _NS
_GCHUNK = 128
_RING_ROWS = 640
_NBUF = 2


def _sc_gather(xf, word_table):
    N = xf.shape[0]
    V, D = word_table.shape
    n_per_w = N // _NW
    n_chunks = n_per_w // _RING_ROWS
    assert n_per_w * _NW == N and n_chunks * _RING_ROWS == n_per_w
    assert n_chunks % _NBUF == 0
    n_gath = _RING_ROWS // _GCHUNK

    mesh = plsc.VectorSubcoreMesh(core_axis_name="c", subcore_axis_name="s")

    @functools.partial(
        pl.kernel,
        out_type=jax.ShapeDtypeStruct((N, D), jnp.float32),
        mesh=mesh,
        scratch_types=[
            pltpu.VMEM((n_per_w,), jnp.int32),
            pltpu.VMEM((_NBUF, _RING_ROWS, D), jnp.float32),
            pltpu.SemaphoreType.DMA,
            pltpu.SemaphoreType.DMA,
            pltpu.SemaphoreType.DMA,
            pltpu.SemaphoreType.DMA,
        ],
        compiler_params=pltpu.CompilerParams(use_tc_tiling_on_sc=False),
    )
    def gather_kernel(x_hbm, word_hbm, out_hbm, idx_v, rows_v,
                      semg0, semg1, semw0, semw1):
        wid = lax.axis_index("s") * _NC + lax.axis_index("c")
        base = wid * n_per_w
        semg = (semg0, semg1)
        semw = (semw0, semw1)

        pltpu.sync_copy(x_hbm.at[pl.ds(base, n_per_w)], idx_v)

        def fire(buf, step):
            off = step * _RING_ROWS
            for j in range(n_gath):
                pltpu.async_copy(
                    word_hbm.at[idx_v.at[pl.ds(off + j * _GCHUNK, _GCHUNK)]],
                    rows_v.at[buf, pl.ds(j * _GCHUNK, _GCHUNK)],
                    semg[buf],
                )

        def drain_g(buf):
            pltpu.make_async_copy(
                word_hbm.at[pl.ds(0, _RING_ROWS)], rows_v.at[buf], semg[buf]
            ).wait()

        def write_async(buf, step):
            pltpu.async_copy(
                rows_v.at[buf],
                out_hbm.at[pl.ds(base + step * _RING_ROWS, _RING_ROWS)],
                semw[buf],
            )

        def wait_w(buf):
            pltpu.make_async_copy(
                rows_v.at[buf], out_hbm.at[pl.ds(0, _RING_ROWS)], semw[buf]
            ).wait()

        fire(0, 0)

        def outer(i, carry):
            s0 = i * _NBUF

            @pl.when(i > 0)
            def _():
                wait_w(1)

            fire(1, s0 + 1)
            drain_g(0)
            write_async(0, s0)

            @pl.when(s0 + 2 < n_chunks)
            def _():
                wait_w(0)
                fire(0, s0 + 2)

            drain_g(1)
            write_async(1, s0 + 1)
            return carry

        lax.fori_loop(0, n_chunks // _NBUF, outer, 0)
        wait_w(0)
        wait_w(1)

    return gather_kernel(xf, word_table)


_SBLK = 2
_P = 4


def _ln_body(g_ref, pos_ref, gamma_ref, beta_ref, *rest):
    o_ref = rest[-1]
    for k in range(_SBLK):
        h = g_ref[k].T + pos_ref[k]
        mean = jnp.mean(h, axis=0, keepdims=True)
        d = h - mean
        var = jnp.mean(d * d, axis=0, keepdims=True)
        o_ref[k] = (
            d * lax.rsqrt(var + jnp.float32(1e-5)) * gamma_ref[...] + beta_ref[...]
        )


def _tc_layernorm_piece(g3, posP, gammaT, betaT, prev, piece, S, B, D):
    Sp = g3.shape[0]
    blk_off = piece * (Sp // _SBLK)
    in_specs = [
        pl.BlockSpec((_SBLK, B, D), lambda s: (s, 0, 0)),
        pl.BlockSpec((_SBLK, D, 1), lambda s: (s, 0, 0)),
        pl.BlockSpec((D, 1), lambda s: (0, 0)),
        pl.BlockSpec((D, 1), lambda s: (0, 0)),
    ]
    args = [g3, posP, gammaT, betaT]
    kwargs = {}
    if prev is not None:
        in_specs.append(pl.BlockSpec((_SBLK, D, B), lambda s: (0, 0, 0)))
        args.append(prev)
        kwargs["input_output_aliases"] = {4: 0}
    return pl.pallas_call(
        _ln_body,
        grid=(Sp // _SBLK,),
        in_specs=in_specs,
        out_specs=pl.BlockSpec((_SBLK, D, B), lambda s: (s + blk_off, 0, 0)),
        out_shape=jax.ShapeDtypeStruct((S, D, B), jnp.float32),
        **kwargs,
    )(*args)


def kernel(x, word_table, pos_table, gamma, beta):
    B, S = x.shape
    V, D = word_table.shape
    N = B * S
    xT = jnp.swapaxes(x, 0, 1).reshape(N)
    posP = pos_table.reshape(S, D, 1)
    gammaT = gamma.reshape(D, 1)
    betaT = beta.reshape(D, 1)
    Sp = S // _P
    out_phys = None
    for p in range(_P):
        xp = xT[p * Sp * B:(p + 1) * Sp * B]
        g = _sc_gather(xp, word_table)
        out_phys = _tc_layernorm_piece(
            g.reshape(Sp, B, D),
            posP[p * Sp:(p + 1) * Sp],
            gammaT,
            betaT,
            out_phys,
            p,
            S, B, D,
        )
    return jnp.transpose(out_phys, (2, 0, 1))
